# baseline (speedup 1.0000x reference)
import functools

import jax
import jax.numpy as jnp
from jax.experimental import pallas as pl
from jax.experimental.pallas import tpu as pltpu

_MAX_DISTANCE = 32


def _fused_fn(tabs_ref, k_ref, v_ref, ko_ref, vo_ref, biast_ref, tt_ref, *, h, hd, s):
    m = s * hd
    n_rows = 2 * _MAX_DISTANCE + 1
    n_cols = tabs_ref.shape[1]
    i = pl.program_id(0)

    @pl.when((i == 0) & (pl.program_id(1) == 0))
    def _build_tables():
        r = jax.lax.broadcasted_iota(jnp.int32, (m, n_rows), 1)
        mm = jax.lax.broadcasted_iota(jnp.int32, (m, n_rows), 0)
        y = mm // hd
        z = mm % hd
        idx = jnp.clip(y - z, -_MAX_DISTANCE, _MAX_DISTANCE) + _MAX_DISTANCE
        onehot = (r == idx).astype(jnp.float32)
        tt = jnp.dot(onehot, tabs_ref[...], preferred_element_type=jnp.float32)
        tt_ref[...] = tt
        biast_ref[...] = tt[:, 2 * h :]

    rr = jax.lax.broadcasted_iota(jnp.int32, (n_cols, 2), 0)
    cc = jax.lax.broadcasted_iota(jnp.int32, (n_cols, 2), 1)
    sel = ((rr == i) & (cc == 0)) | ((rr == i + h) & (cc == 1))
    cols = jnp.dot(
        tt_ref[...], sel.astype(jnp.float32), preferred_element_type=jnp.float32
    )
    ko_ref[...] = k_ref[...] + cols[None, :, 0:1]
    vo_ref[...] = v_ref[...] + cols[None, :, 1:2]


@jax.jit
def kernel(q, k, v, bias_table, embed_k, embed_v):
    del q
    B, H, S, HD = k.shape
    M = S * HD
    N_ROWS = 2 * _MAX_DISTANCE + 1

    kt = k.transpose(1, 2, 3, 0).reshape(H, M, B)
    vt = v.transpose(1, 2, 3, 0).reshape(H, M, B)

    tabs = jnp.concatenate(
        [
            embed_k,
            embed_v,
            jnp.pad(bias_table, ((0, 1), (0, 0))),
        ],
        axis=1,
    )

    NB = 2
    BB = B // NB
    grid = (H, NB)
    ko, vo, biast = pl.pallas_call(
        functools.partial(_fused_fn, h=H, hd=HD, s=S),
        grid=grid,
        in_specs=[
            pl.BlockSpec((N_ROWS, 3 * H), lambda i, j: (0, 0)),
            pl.BlockSpec((1, M, BB), lambda i, j: (i, 0, j)),
            pl.BlockSpec((1, M, BB), lambda i, j: (i, 0, j)),
        ],
        out_specs=[
            pl.BlockSpec((1, M, BB), lambda i, j: (i, 0, j)),
            pl.BlockSpec((1, M, BB), lambda i, j: (i, 0, j)),
            pl.BlockSpec((M, H), lambda i, j: (0, 0)),
        ],
        out_shape=[
            jax.ShapeDtypeStruct((H, M, B), jnp.float32),
            jax.ShapeDtypeStruct((H, M, B), jnp.float32),
            jax.ShapeDtypeStruct((M, H), jnp.float32),
        ],
        scratch_shapes=[pltpu.VMEM((M, 3 * H), jnp.float32)],
    )(tabs, kt, vt)

    k_out = ko.reshape(H, S, HD, B).transpose(3, 0, 1, 2)
    v_out = vo.reshape(H, S, HD, B).transpose(3, 0, 1, 2)
    bias = biast.T.reshape(H, S, S)
    return (k_out, v_out, bias)
